# baseline (speedup 1.0000x reference)
import functools
import math

import jax
import jax.numpy as jnp
import numpy as np
from jax import lax
from jax.experimental import pallas as pl
from jax.experimental.pallas import tpu as pltpu
from jax.experimental.pallas import tpu_sc as plsc

VOCAB = 1000000
EMB_DIM = 64
BATCH = 4
SEQ = 2048
SCALE = math.sqrt(EMB_DIM)

NC, NS, L = 2, 16, 16
NW = NC * NS
B_TOTAL = BATCH * SEQ
B_PER_W = B_TOTAL // NW
PE_CHUNKS = SEQ // B_PER_W


def _sinusoidal_pe(seq_len, d):
    pos = np.arange(seq_len, dtype=np.float32)[:, None]
    div = np.exp(np.arange(0, d, 2, dtype=np.float32) * (-math.log(10000.0) / d))
    pe = np.zeros((seq_len, d), dtype=np.float32)
    pe[:, 0::2] = np.sin(pos * div)
    pe[:, 1::2] = np.cos(pos * div)
    return pe

_PE = _sinusoidal_pe(SEQ, EMB_DIM)


def _build_sc_kernel():
    mesh = plsc.VectorSubcoreMesh(core_axis_name="c", subcore_axis_name="s",
                                  num_cores=NC, num_subcores=NS)

    @functools.partial(
        pl.kernel,
        out_type=jax.ShapeDtypeStruct((B_TOTAL, EMB_DIM), jnp.float32),
        mesh=mesh,
        scratch_types=[
            pltpu.VMEM((B_PER_W,), jnp.int32),
            pltpu.VMEM((B_PER_W, EMB_DIM), jnp.float32),
            pltpu.VMEM((B_PER_W, EMB_DIM), jnp.float32),
            pltpu.SemaphoreType.DMA,
        ],
        compiler_params=pltpu.CompilerParams(use_tc_tiling_on_sc=False),
    )
    def emb_kernel(idx_hbm, pe_hbm, table_hbm, out_hbm, idx_v, rows_v, pe_v, sem):
        wid = lax.axis_index("s") * NC + lax.axis_index("c")
        base = wid * B_PER_W
        pltpu.sync_copy(idx_hbm.at[pl.ds(base, B_PER_W)], idx_v)
        gather = pltpu.async_copy(table_hbm.at[idx_v], rows_v, sem)
        pe_base = lax.rem(wid, PE_CHUNKS) * B_PER_W
        pltpu.sync_copy(pe_hbm.at[pl.ds(pe_base, B_PER_W)], pe_v)
        gather.wait()

        def body(r, _):
            for d in range(EMB_DIM // L):
                sl = pl.ds(d * L, L)
                rows_v[r, sl] = rows_v[r, sl] * SCALE + pe_v[r, sl]
            return _

        lax.fori_loop(0, B_PER_W, body, None)
        pltpu.sync_copy(rows_v, out_hbm.at[pl.ds(base, B_PER_W)])

    return emb_kernel


def kernel(x, tok_emb):
    idx = x.reshape(-1).astype(jnp.int32)
    out = _build_sc_kernel()(idx, _PE, tok_emb)
    return out.reshape(BATCH, SEQ, EMB_DIM)
